# baseline (speedup 1.0000x reference)
import functools

import jax
import jax.numpy as jnp
from jax import lax
from jax.experimental import pallas as pl
from jax.experimental.pallas import tpu as pltpu
from jax.experimental.pallas import tpu_sc as plsc

_N_CORES = 2
_N_SUB = 16


def _tile_kernel(col_ref, row_ref, o_ref):
    nc = col_ref.shape[1]
    w = col_ref.shape[0]
    h = row_ref.shape[0]
    col_t = col_ref[...].T
    row_t = row_ref[...].T
    o_ref[:nc] = jnp.broadcast_to(col_t[:, None, :], (nc, h, w)).reshape(nc, h * w)
    o_ref[nc:] = jnp.broadcast_to(row_t[:, :, None], (nc, h, w)).reshape(nc, h * w)


def _bcast_sc_kernel(tile_hbm, o_hbm, shared, sem):
    cid = lax.axis_index("c")
    sid = lax.axis_index("s")
    wid = sid * _N_CORES + cid

    n = tile_hbm.shape[0]
    chunk = n // _N_SUB
    base = sid * chunk
    pltpu.sync_copy(
        tile_hbm.at[pl.ds(base, chunk)], shared.at[pl.ds(base, chunk)]
    )
    plsc.subcore_barrier()

    pltpu.async_copy(shared, o_hbm.at[wid], sem).wait()


def kernel(x, row_w, col_w):
    b = x.shape[0]
    h, w = x.shape[-2], x.shape[-1]
    nc = row_w.shape[1]
    flat = 2 * nc * h * w

    tile = pl.pallas_call(
        _tile_kernel,
        out_shape=jax.ShapeDtypeStruct((2 * nc, h * w), jnp.float32),
    )(col_w, row_w)

    mesh = plsc.VectorSubcoreMesh(core_axis_name="c", subcore_axis_name="s")
    run = functools.partial(
        pl.kernel,
        out_type=jax.ShapeDtypeStruct((b, flat), jnp.float32),
        mesh=mesh,
        scratch_types=[
            pltpu.VMEM_SHARED((flat,), jnp.float32),
            pltpu.SemaphoreType.DMA,
        ],
    )(_bcast_sc_kernel)
    out = run(tile.reshape(flat))
    return out.reshape(b, 2 * nc, h, w)


# speedup vs baseline: 1.0013x; 1.0013x over previous
import functools

import jax
import jax.numpy as jnp
from jax import lax
from jax.experimental import pallas as pl
from jax.experimental.pallas import tpu as pltpu
from jax.experimental.pallas import tpu_sc as plsc

_N_CORES = 2
_N_SUB = 16


def _tile_kernel(col_ref, row_ref, o_ref):
    nc = col_ref.shape[1]
    w = col_ref.shape[0]
    h = row_ref.shape[0]
    col_t = col_ref[...].T
    row_t = row_ref[...].T
    o_ref[:nc] = jnp.broadcast_to(col_t[:, None, :], (nc, h, w)).reshape(nc, h * w)
    o_ref[nc:] = jnp.broadcast_to(row_t[:, :, None], (nc, h, w)).reshape(nc, h * w)


def _bcast_sc_kernel(tile_hbm, o_hbm, shared, sem):
    cid = lax.axis_index("c")
    sid = lax.axis_index("s")
    wid = sid * _N_CORES + cid

    n = tile_hbm.shape[0]
    chunk = n // _N_SUB
    base = sid * chunk
    pltpu.sync_copy(
        tile_hbm.at[pl.ds(base, chunk)], shared.at[pl.ds(base, chunk)]
    )
    plsc.subcore_barrier()

    pltpu.async_copy(shared, o_hbm.at[wid], sem).wait()


def kernel(x, row_w, col_w):
    b = x.shape[0]
    h, w = x.shape[-2], x.shape[-1]
    nc = row_w.shape[1]
    flat = 2 * nc * h * w

    tile = pl.pallas_call(
        _tile_kernel,
        out_shape=jax.ShapeDtypeStruct((2 * nc, h * w), jnp.float32),
    )(col_w, row_w)

    mesh = plsc.VectorSubcoreMesh(core_axis_name="c", subcore_axis_name="s")
    run = functools.partial(
        pl.kernel,
        out_type=jax.ShapeDtypeStruct((b, flat), jnp.float32),
        mesh=mesh,
        scratch_types=[
            pltpu.VMEM_SHARED((flat,), jnp.float32),
            pltpu.SemaphoreType.DMA,
        ],
        compiler_params=pltpu.CompilerParams(use_tc_tiling_on_sc=True),
    )(_bcast_sc_kernel)
    out = run(tile.reshape(flat))
    return out.reshape(b, 2 * nc, h, w)


# speedup vs baseline: 2.6860x; 2.6825x over previous
import functools

import jax
import jax.numpy as jnp
from jax import lax
from jax.experimental import pallas as pl
from jax.experimental.pallas import tpu as pltpu
from jax.experimental.pallas import tpu_sc as plsc

_N_CORES = 2
_N_SUB = 16


def _tile_kernel(col_ref, row_ref, o_ref):
    nc = col_ref.shape[1]
    w = col_ref.shape[0]
    h = row_ref.shape[0]
    col_t = col_ref[...].T
    row_t = row_ref[...].T
    o_ref[:nc] = jnp.broadcast_to(col_t[:, None, :], (nc, h, w)).reshape(nc, h * w)
    o_ref[nc:] = jnp.broadcast_to(row_t[:, :, None], (nc, h, w)).reshape(nc, h * w)


def _bcast_sc_kernel(tile_hbm, o_hbm, shared, sem):
    cid = lax.axis_index("c")
    sid = lax.axis_index("s")
    wid = sid * _N_CORES + cid

    n = tile_hbm.shape[0]
    chunk = n // _N_SUB
    base = sid * chunk
    pltpu.sync_copy(
        tile_hbm.at[pl.ds(base, chunk)], shared.at[pl.ds(base, chunk)]
    )
    plsc.subcore_barrier()

    pltpu.async_copy(shared, o_hbm.at[wid], sem).wait()


_UNUSED = None


def kernel(x, row_w, col_w):
    b = x.shape[0]
    h, w = x.shape[-2], x.shape[-1]
    nc = row_w.shape[1]
    flat = 2 * nc * h * w

    tile = pl.pallas_call(
        _tile_kernel,
        out_shape=jax.ShapeDtypeStruct((2 * nc, h * w), jnp.float32),
    )(col_w, row_w)

    mesh = plsc.VectorSubcoreMesh(core_axis_name="c", subcore_axis_name="s")
    run = functools.partial(
        pl.kernel,
        out_type=jax.ShapeDtypeStruct((b, 2 * nc, h * w), jnp.float32),
        mesh=mesh,
        scratch_types=[
            pltpu.VMEM_SHARED((2 * nc, h * w), jnp.float32),
            pltpu.SemaphoreType.DMA,
        ],
        compiler_params=pltpu.CompilerParams(use_tc_tiling_on_sc=True),
    )(_bcast_sc_kernel)
    out = run(tile)
    return out.reshape(b, 2 * nc, h, w)


# speedup vs baseline: 4.3361x; 1.6143x over previous
import jax
import jax.numpy as jnp
from jax.experimental import pallas as pl
from jax.experimental.pallas import tpu as pltpu


def _pos_kernel(col_ref, row_ref, o_hbm, scratch, sem):
    nc = col_ref.shape[1]
    w = col_ref.shape[0]
    h = row_ref.shape[0]
    col_t = col_ref[...].T
    row_t = row_ref[...].T
    scratch[:nc] = jnp.broadcast_to(col_t[:, None, :], (nc, h, w)).reshape(nc, h * w)
    scratch[nc:] = jnp.broadcast_to(row_t[:, :, None], (nc, h, w)).reshape(nc, h * w)
    b_total = o_hbm.shape[0]
    for b in range(b_total):
        pltpu.make_async_copy(scratch, o_hbm.at[b], sem).start()
    for b in range(b_total):
        pltpu.make_async_copy(scratch, o_hbm.at[b], sem).wait()


def kernel(x, row_w, col_w):
    b = x.shape[0]
    h, w = x.shape[-2], x.shape[-1]
    nc = row_w.shape[1]
    out = pl.pallas_call(
        _pos_kernel,
        in_specs=[
            pl.BlockSpec(memory_space=pltpu.MemorySpace.VMEM),
            pl.BlockSpec(memory_space=pltpu.MemorySpace.VMEM),
        ],
        out_specs=pl.BlockSpec(memory_space=pl.ANY),
        out_shape=jax.ShapeDtypeStruct((b, 2 * nc, h * w), jnp.float32),
        scratch_shapes=[
            pltpu.VMEM((2 * nc, h * w), jnp.float32),
            pltpu.SemaphoreType.DMA,
        ],
    )(col_w, row_w)
    return out.reshape(b, 2 * nc, h, w)
